# baseline (speedup 1.0000x reference)
import functools

import jax
import jax.numpy as jnp
from jax.experimental import pallas as pl

M, B, D = 16, 8192, 128
E = 16
NUM_CLASSES = 2
T = B // 2

_NEG = float(-3.4e38)


def _argmax16(v, srow16):
    idx = srow16
    r = E
    while r > 1:
        h = r // 2
        take = v[h:r, :] > v[:h, :]
        v = jnp.where(take, v[h:r, :], v[:h, :])
        idx = jnp.where(take, idx[h:r, :], idx[:h, :])
        r = h
    return v, idx


def _routed_vals(x_blk, wcat, addvec, srow16):
    z = jax.lax.dot_general(
        wcat, x_blk,
        dimension_numbers=(((1,), (1,)), ((), ())),
        preferred_element_type=jnp.float32,
    ) + addvec

    logits = z[:E, :]
    _, a1 = _argmax16(logits, srow16)
    logits2 = jnp.where(srow16 == a1, _NEG, logits)
    _, a2 = _argmax16(logits2, srow16)
    estar = jnp.maximum(a1, a2)

    v = z[E:, :]
    for bit in (3, 2, 1, 0):
        h = v.shape[0] // 2
        take = (estar & (1 << bit)) != 0
        v = jnp.where(take, v[h:, :], v[:h, :])
    return v


def _fused_kernel(xa_ref, xb_ref, wcat_ref, addvec_ref, out_ref):
    i = pl.program_id(0)
    wcat = wcat_ref[...]
    addvec = addvec_ref[...]
    srow16 = jax.lax.broadcasted_iota(jnp.int32, (E, T), 0)
    scale = float(1.0 / M)

    @pl.when(i == 0)
    def _init():
        out_ref[...] = jnp.zeros((8, B), jnp.float32)

    va = _routed_vals(xa_ref[...], wcat, addvec, srow16) * scale
    vb = _routed_vals(xb_ref[...], wcat, addvec, srow16) * scale
    out_ref[0:2, 0:T] += va
    out_ref[0:2, T:B] += vb


@functools.partial(jax.jit, static_argnames=())
def kernel(x, gate_W, gate_b, expert_W, expert_b, head_W, head_b):
    V = jnp.einsum("cd,edf->ecf", head_W, expert_W)
    Vflat = V.reshape(E * NUM_CLASSES, D)
    wcat = jnp.concatenate([gate_W, Vflat], axis=0)
    ce = expert_b @ head_W.T + head_b[None, :]
    addvec = jnp.concatenate([gate_b, ce.reshape(-1)])[:, None]

    xflat = x.reshape(M * B, D)
    out_padded = pl.pallas_call(
        _fused_kernel,
        grid=(M,),
        in_specs=[
            pl.BlockSpec((T, D), lambda i: (2 * i, 0)),
            pl.BlockSpec((T, D), lambda i: (2 * i + 1, 0)),
            pl.BlockSpec((3 * E, D), lambda i: (0, 0)),
            pl.BlockSpec((3 * E, 1), lambda i: (0, 0)),
        ],
        out_specs=pl.BlockSpec((8, B), lambda i: (0, 0)),
        out_shape=jax.ShapeDtypeStruct((8, B), jnp.float32),
    )(xflat, xflat, wcat, addvec)
    return out_padded[:NUM_CLASSES, :].T


# speedup vs baseline: 1.1157x; 1.1157x over previous
import functools

import jax
import jax.numpy as jnp
from jax.experimental import pallas as pl

M, B, D = 16, 8192, 128
E = 16
NUM_CLASSES = 2
T = 2 * B

_NEG = float(-3.4e38)


def _argmax16(v, srow16):
    idx = srow16
    r = E
    while r > 1:
        h = r // 2
        take = v[h:r, :] > v[:h, :]
        v = jnp.where(take, v[h:r, :], v[:h, :])
        idx = jnp.where(take, idx[h:r, :], idx[:h, :])
        r = h
    return v, idx


def _routed_vals(x_blk, wcat, addvec, srow16):
    z = jax.lax.dot_general(
        wcat, x_blk,
        dimension_numbers=(((1,), (1,)), ((), ())),
        preferred_element_type=jnp.float32,
    ) + addvec

    logits = z[:E, :]
    _, a1 = _argmax16(logits, srow16)
    logits2 = jnp.where(srow16 == a1, _NEG, logits)
    _, a2 = _argmax16(logits2, srow16)
    estar = jnp.maximum(a1, a2)

    v = z[E:, :]
    for bit in (3, 2, 1, 0):
        h = v.shape[0] // 2
        take = (estar & (1 << bit)) != 0
        v = jnp.where(take, v[h:, :], v[:h, :])
    return v


def _fused_kernel(xa_ref, xb_ref, wcat_ref, addvec_ref, out_ref):
    i = pl.program_id(0)
    wcat = wcat_ref[...]
    addvec = addvec_ref[...]
    srow16 = jax.lax.broadcasted_iota(jnp.int32, (E, T), 0)
    scale = float(1.0 / M)

    @pl.when(i == 0)
    def _init():
        out_ref[...] = jnp.zeros((8, B), jnp.float32)

    va = _routed_vals(xa_ref[...], wcat, addvec, srow16)
    vb = _routed_vals(xb_ref[...], wcat, addvec, srow16)
    out_ref[0:2, :] += (va[:, 0:B] + va[:, B:2 * B]
                        + vb[:, 0:B] + vb[:, B:2 * B]) * scale


@functools.partial(jax.jit, static_argnames=())
def kernel(x, gate_W, gate_b, expert_W, expert_b, head_W, head_b):
    V = jnp.einsum("cd,edf->ecf", head_W, expert_W)
    Vflat = V.reshape(E * NUM_CLASSES, D)
    wcat = jnp.concatenate([gate_W, Vflat], axis=0)
    ce = expert_b @ head_W.T + head_b[None, :]
    addvec = jnp.concatenate([gate_b, ce.reshape(-1)])[:, None]

    xflat = x.reshape(M * B, D)
    out_padded = pl.pallas_call(
        _fused_kernel,
        grid=(M * B // (2 * T),),
        in_specs=[
            pl.BlockSpec((T, D), lambda i: (2 * i, 0)),
            pl.BlockSpec((T, D), lambda i: (2 * i + 1, 0)),
            pl.BlockSpec((3 * E, D), lambda i: (0, 0)),
            pl.BlockSpec((3 * E, 1), lambda i: (0, 0)),
        ],
        out_specs=pl.BlockSpec((8, B), lambda i: (0, 0)),
        out_shape=jax.ShapeDtypeStruct((8, B), jnp.float32),
    )(xflat, xflat, wcat, addvec)
    return out_padded[:NUM_CLASSES, :].T


# speedup vs baseline: 1.1170x; 1.0011x over previous
import functools

import jax
import jax.numpy as jnp
from jax.experimental import pallas as pl

M, B, D = 16, 8192, 128
E = 16
NUM_CLASSES = 2
T = B

_NEG = float(-3.4e38)


def _argmax16(v, srow16):
    idx = srow16
    r = E
    while r > 1:
        h = r // 2
        take = v[h:r, :] > v[:h, :]
        v = jnp.where(take, v[h:r, :], v[:h, :])
        idx = jnp.where(take, idx[h:r, :], idx[:h, :])
        r = h
    return v, idx


def _routed_vals(x_blk, wcat, addvec, srow16):
    z = jax.lax.dot_general(
        wcat, x_blk,
        dimension_numbers=(((1,), (1,)), ((), ())),
        preferred_element_type=jnp.float32,
    ) + addvec

    logits = z[:E, :]
    _, a1 = _argmax16(logits, srow16)
    logits2 = jnp.where(srow16 == a1, _NEG, logits)
    _, a2 = _argmax16(logits2, srow16)
    estar = jnp.maximum(a1, a2)

    v = z[E:, :]
    for bit in (3, 2, 1, 0):
        h = v.shape[0] // 2
        take = (estar & (1 << bit)) != 0
        v = jnp.where(take, v[h:, :], v[:h, :])
    return v


def _fused_kernel(xa_ref, xb_ref, wcat_ref, addvec_ref, out_ref):
    i = pl.program_id(0)
    wcat = wcat_ref[...]
    addvec = addvec_ref[...]
    srow16 = jax.lax.broadcasted_iota(jnp.int32, (E, T), 0)
    scale = float(1.0 / M)

    @pl.when(i == 0)
    def _init():
        out_ref[...] = jnp.zeros((8, B), jnp.float32)

    va = _routed_vals(xa_ref[...], wcat, addvec, srow16)
    vb = _routed_vals(xb_ref[...], wcat, addvec, srow16)
    out_ref[0:2, :] += (va + vb) * scale


@functools.partial(jax.jit, static_argnames=())
def kernel(x, gate_W, gate_b, expert_W, expert_b, head_W, head_b):
    V = jnp.einsum("cd,edf->ecf", head_W, expert_W)
    Vflat = V.reshape(E * NUM_CLASSES, D)
    wcat = jnp.concatenate([gate_W, Vflat], axis=0)
    ce = expert_b @ head_W.T + head_b[None, :]
    addvec = jnp.concatenate([gate_b, ce.reshape(-1)])[:, None]

    xflat = x.reshape(M * B, D)
    out_padded = pl.pallas_call(
        _fused_kernel,
        grid=(M * B // (2 * T),),
        in_specs=[
            pl.BlockSpec((T, D), lambda i: (2 * i, 0)),
            pl.BlockSpec((T, D), lambda i: (2 * i + 1, 0)),
            pl.BlockSpec((3 * E, D), lambda i: (0, 0)),
            pl.BlockSpec((3 * E, 1), lambda i: (0, 0)),
        ],
        out_specs=pl.BlockSpec((8, B), lambda i: (0, 0)),
        out_shape=jax.ShapeDtypeStruct((8, B), jnp.float32),
    )(xflat, xflat, wcat, addvec)
    return out_padded[:NUM_CLASSES, :].T


# speedup vs baseline: 1.1284x; 1.0102x over previous
import functools

import jax
import jax.numpy as jnp
from jax.experimental import pallas as pl
from jax.experimental.pallas import tpu as pltpu

M, B, D = 16, 8192, 128
E = 16
NUM_CLASSES = 2
NBUF = 4

_NEG = float(-3.4e38)


def _argmax16(v, srow16):
    idx = srow16
    r = E
    while r > 1:
        h = r // 2
        take = v[h:r, :] > v[:h, :]
        v = jnp.where(take, v[h:r, :], v[:h, :])
        idx = jnp.where(take, idx[h:r, :], idx[:h, :])
        r = h
    return v, idx


def _routed_vals(x_blk, wcat, addvec, srow16):
    z = jax.lax.dot_general(
        wcat, x_blk,
        dimension_numbers=(((1,), (1,)), ((), ())),
        preferred_element_type=jnp.float32,
    ) + addvec

    logits = z[:E, :]
    _, a1 = _argmax16(logits, srow16)
    logits2 = jnp.where(srow16 == a1, _NEG, logits)
    _, a2 = _argmax16(logits2, srow16)
    estar = jnp.maximum(a1, a2)

    v = z[E:, :]
    for bit in (3, 2, 1, 0):
        h = v.shape[0] // 2
        take = (estar & (1 << bit)) != 0
        v = jnp.where(take, v[h:, :], v[:h, :])
    return v


def _copy(x_hbm, xbuf, sem, k):
    slot = k % NBUF
    return pltpu.make_async_copy(
        x_hbm.at[k], xbuf.at[slot], sem.at[slot])


def _fused_kernel(x_ref, wcat_ref, addvec_ref, out_ref, xbuf, sem):
    wcat = wcat_ref[...]
    addvec = addvec_ref[...]
    srow16 = jax.lax.broadcasted_iota(jnp.int32, (E, B), 0)
    scale = float(1.0 / M)

    for k in range(NBUF):
        _copy(x_ref, xbuf, sem, k).start()

    acc = jnp.zeros((2, B), jnp.float32)
    for k in range(M):
        _copy(x_ref, xbuf, sem, k).wait()
        acc += _routed_vals(xbuf[k % NBUF], wcat, addvec, srow16)
        if k + NBUF < M:
            _copy(x_ref, xbuf, sem, k + NBUF).start()

    orow = jax.lax.broadcasted_iota(jnp.int32, (8, B), 0)
    out_ref[...] = jnp.where(orow == 0, acc[0:1, :] * scale,
                             jnp.where(orow == 1, acc[1:2, :] * scale, 0.0))


@functools.partial(jax.jit, static_argnames=())
def kernel(x, gate_W, gate_b, expert_W, expert_b, head_W, head_b):
    V = jnp.einsum("cd,edf->ecf", head_W, expert_W)
    Vflat = V.reshape(E * NUM_CLASSES, D)
    wcat = jnp.concatenate([gate_W, Vflat], axis=0)
    ce = expert_b @ head_W.T + head_b[None, :]
    addvec = jnp.concatenate([gate_b, ce.reshape(-1)])[:, None]

    out_padded = pl.pallas_call(
        _fused_kernel,
        in_specs=[
            pl.BlockSpec(memory_space=pl.ANY),
            pl.BlockSpec(memory_space=pltpu.MemorySpace.VMEM),
            pl.BlockSpec(memory_space=pltpu.MemorySpace.VMEM),
        ],
        out_specs=pl.BlockSpec(memory_space=pltpu.MemorySpace.VMEM),
        out_shape=jax.ShapeDtypeStruct((8, B), jnp.float32),
        scratch_shapes=[
            pltpu.VMEM((NBUF, B, D), jnp.float32),
            pltpu.SemaphoreType.DMA((NBUF,)),
        ],
    )(x.reshape(M, B, D), wcat, addvec)
    return out_padded[:NUM_CLASSES, :].T


# speedup vs baseline: 1.1326x; 1.0037x over previous
import functools

import jax
import jax.numpy as jnp
from jax.experimental import pallas as pl
from jax.experimental.pallas import tpu as pltpu

M, B, D = 16, 8192, 128
E = 16
NUM_CLASSES = 2
NBUF = 4

_NEG = float(-3.4e38)


def _argmax16(v, srow16):
    idx = srow16
    r = E
    while r > 1:
        h = r // 2
        take = v[h:r, :] > v[:h, :]
        v = jnp.where(take, v[h:r, :], v[:h, :])
        idx = jnp.where(take, idx[h:r, :], idx[:h, :])
        r = h
    return v, idx


def _routed_vals(x_blk, wcat, addvec, srow16):
    z = jax.lax.dot_general(
        wcat, x_blk,
        dimension_numbers=(((1,), (1,)), ((), ())),
        preferred_element_type=jnp.float32,
    ) + addvec

    logits = z[:E, :]
    _, a1 = _argmax16(logits, srow16)
    logits2 = jnp.where(srow16 == a1, _NEG, logits)
    _, a2 = _argmax16(logits2, srow16)
    estar = jnp.maximum(a1, a2)

    v = z[E:, :]
    for bit in (3, 2, 1, 0):
        h = v.shape[0] // 2
        take = (estar & (1 << bit)) != 0
        v = jnp.where(take, v[h:, :], v[:h, :])
    return v


def _copy(x_hbm, xbuf, sem, k):
    slot = k % NBUF
    return pltpu.make_async_copy(
        x_hbm.at[k], xbuf.at[slot], sem.at[slot])


def _fused_kernel(x_ref, wcat_ref, addvec_ref, out_ref, xbuf, sem):
    wcat = wcat_ref[...]
    addvec = addvec_ref[...]
    srow16 = jax.lax.broadcasted_iota(jnp.int32, (E, B), 0)
    scale = float(1.0 / M)

    for k in range(NBUF):
        _copy(x_ref, xbuf, sem, k).start()

    SW = 1024
    acc = jnp.zeros((2, B), jnp.float32)
    for k in range(M):
        _copy(x_ref, xbuf, sem, k).wait()
        strips = []
        for s in range(B // SW):
            xs = xbuf[k % NBUF, s * SW:(s + 1) * SW, :]
            strips.append(_routed_vals(xs, wcat, addvec, srow16[:, :SW]))
        acc += jnp.concatenate(strips, axis=1)
        if k + NBUF < M:
            _copy(x_ref, xbuf, sem, k + NBUF).start()

    orow = jax.lax.broadcasted_iota(jnp.int32, (8, B), 0)
    out_ref[...] = jnp.where(orow == 0, acc[0:1, :] * scale,
                             jnp.where(orow == 1, acc[1:2, :] * scale, 0.0))


@functools.partial(jax.jit, static_argnames=())
def kernel(x, gate_W, gate_b, expert_W, expert_b, head_W, head_b):
    V = jnp.einsum("cd,edf->ecf", head_W, expert_W)
    Vflat = V.reshape(E * NUM_CLASSES, D)
    wcat = jnp.concatenate([gate_W, Vflat], axis=0)
    ce = expert_b @ head_W.T + head_b[None, :]
    addvec = jnp.concatenate([gate_b, ce.reshape(-1)])[:, None]

    out_padded = pl.pallas_call(
        _fused_kernel,
        in_specs=[
            pl.BlockSpec(memory_space=pl.ANY),
            pl.BlockSpec(memory_space=pltpu.MemorySpace.VMEM),
            pl.BlockSpec(memory_space=pltpu.MemorySpace.VMEM),
        ],
        out_specs=pl.BlockSpec(memory_space=pltpu.MemorySpace.VMEM),
        out_shape=jax.ShapeDtypeStruct((8, B), jnp.float32),
        scratch_shapes=[
            pltpu.VMEM((NBUF, B, D), jnp.float32),
            pltpu.SemaphoreType.DMA((NBUF,)),
        ],
    )(x.reshape(M, B, D), wcat, addvec)
    return out_padded[:NUM_CLASSES, :].T
